# baseline (speedup 1.0000x reference)
import functools
import numpy as np
import jax
import jax.numpy as jnp
from jax import lax
from jax.experimental import pallas as pl
from jax.experimental.pallas import tpu as pltpu
from jax.experimental.pallas import tpu_sc as plsc

DIM = 32
SCALE = np.sqrt(np.float32(DIM)).astype(np.float32)
NC, NS = 2, 16
NW = NC * NS
NB2 = 2
NBUF = 4
GATHER_AHEAD = 3


@functools.lru_cache(maxsize=None)
def _make(B1, B2):
    cols_w = B1 // NW
    n_chunks = B2 // NB2
    n_groups = n_chunks // NBUF
    assert B2 % NB2 == 0 and n_chunks % NBUF == 0
    mesh = plsc.VectorSubcoreMesh(
        core_axis_name="c", subcore_axis_name="s",
        num_cores=NC, num_subcores=NS)

    @functools.partial(
        pl.kernel,
        out_type=jax.ShapeDtypeStruct((B2, DIM, B1), jnp.float32),
        mesh=mesh,
        scratch_types=(
            [pltpu.VMEM((B2, cols_w), jnp.int32)]
            + [pltpu.VMEM((NB2 * cols_w, DIM), jnp.float32)] * NBUF
            + [pltpu.VMEM((NB2, DIM, cols_w + 1), jnp.float32)] * NBUF
            + [pltpu.SemaphoreType.DMA] * (2 * NBUF)
        ),
        compiler_params=pltpu.CompilerParams(
            use_tc_tiling_on_sc=False, needs_layout_passes=False),
    )
    def emb_kernel(table_hbm, tt_hbm, xt_hbm, out_hbm, idx_v, *scratch):
        del tt_hbm
        gbufs = scratch[:NBUF]
        obufs = scratch[NBUF:2 * NBUF]
        gsems = scratch[2 * NBUF:3 * NBUF]
        ssems = scratch[3 * NBUF:]
        wid = lax.axis_index("s") * NC + lax.axis_index("c")
        col0 = wid * cols_w

        def fire_gather(c, b):
            for s in range(NB2):
                pltpu.async_copy(
                    table_hbm.at[idx_v.at[c * NB2 + s]],
                    gbufs[b].at[pl.ds(s * cols_w, cols_w)],
                    gsems[b])

        def wait_gather(b):
            pltpu.make_async_copy(
                table_hbm.at[pl.ds(0, NB2 * cols_w)],
                gbufs[b], gsems[b]).wait()

        def fire_scatter(c, b):
            pltpu.async_copy(
                obufs[b].at[:, :, pl.ds(0, cols_w)],
                out_hbm.at[pl.ds(c * NB2, NB2), :, pl.ds(col0, cols_w)],
                ssems[b])

        def wait_scatter(b):
            pltpu.make_async_copy(
                obufs[b].at[:, :, pl.ds(0, cols_w)],
                out_hbm.at[pl.ds(0, NB2), :, pl.ds(0, cols_w)],
                ssems[b]).wait()

        lane = lax.iota(jnp.int32, 16)
        dim_rows = [lane + 16 * h for h in range(DIM // 16)]
        s_ids = [jnp.broadcast_to(jnp.int32(s), (16,)) for s in range(NB2)]

        def scale(b):
            gbuf, obuf = gbufs[b], obufs[b]

            @plsc.parallel_loop(0, cols_w, step=1, unroll=4)
            def _scale(l):
                lcol = jnp.broadcast_to(l, (16,))
                for s in range(NB2):
                    for h in range(DIM // 16):
                        vec = gbuf[s * cols_w + l, pl.ds(16 * h, 16)]
                        plsc.store_scatter(
                            obuf, [s_ids[s], dim_rows[h], lcol], vec * SCALE)

        pltpu.sync_copy(xt_hbm.at[:, pl.ds(col0, cols_w)], idx_v)

        for c in range(GATHER_AHEAD):
            fire_gather(c, c % NBUF)

        @pl.loop(0, n_groups)
        def _group(g):
            for i in range(NBUF):
                c = g * NBUF + i
                wait_gather(i)

                @pl.when(c >= NBUF)
                def _():
                    wait_scatter(i)

                scale(i)
                fire_scatter(c, i)

                @pl.when(c + GATHER_AHEAD < n_chunks)
                def _():
                    fire_gather(c + GATHER_AHEAD, (i + GATHER_AHEAD) % NBUF)

        for c in range(n_chunks - NBUF, n_chunks):
            wait_scatter(c % NBUF)

    return emb_kernel


def kernel(x, table):
    B1, B2 = x.shape
    xt = jnp.transpose(x, (1, 0)).astype(jnp.int32)
    tt = jnp.transpose(table, (1, 0))
    out = _make(B1, B2)(table, tt, xt)
    return jnp.transpose(out, (2, 0, 1))


# speedup vs baseline: 3.3895x; 3.3895x over previous
import functools
import numpy as np
import jax
import jax.numpy as jnp
from jax import lax
from jax.experimental import pallas as pl
from jax.experimental.pallas import tpu as pltpu
from jax.experimental.pallas import tpu_sc as plsc

DIM = 32
SCALE = np.sqrt(np.float32(DIM)).astype(np.float32)
NC, NS = 2, 16
NW = NC * NS
ROWPACK = 128 // DIM
NBUF = 4
GATHER_AHEAD = 3


@functools.lru_cache(maxsize=None)
def _make(B1, B2):
    cols_w = B1 // NW
    n_groups = B2 // NBUF
    mesh = plsc.VectorSubcoreMesh(
        core_axis_name="c", subcore_axis_name="s",
        num_cores=NC, num_subcores=NS)

    @functools.partial(
        pl.kernel,
        out_type=jax.ShapeDtypeStruct((B2, DIM, B1), jnp.float32),
        mesh=mesh,
        scratch_types=(
            [pltpu.VMEM((B2, cols_w), jnp.int32)]
            + [pltpu.VMEM((NBUF, cols_w), jnp.int32)]
            + [pltpu.VMEM((cols_w, 128), jnp.float32)] * NBUF
            + [pltpu.VMEM((DIM, cols_w + 1), jnp.float32)] * NBUF
            + [pltpu.SemaphoreType.DMA] * (2 * NBUF)
        ),
        compiler_params=pltpu.CompilerParams(
            use_tc_tiling_on_sc=True, needs_layout_passes=False),
    )
    def emb_kernel(t4_hbm, xt_hbm, out_hbm, idx_v, idx4_v, *scratch):
        gbufs = scratch[:NBUF]
        obufs = scratch[NBUF:2 * NBUF]
        gsems = scratch[2 * NBUF:3 * NBUF]
        ssems = scratch[3 * NBUF:]
        wid = lax.axis_index("s") * NC + lax.axis_index("c")
        col0 = wid * cols_w

        def fire_gather(c, b):
            for g in range(cols_w // 16):
                sl = pl.ds(16 * g, 16)
                idx4_v[b, sl] = lax.shift_right_logical(idx_v[c, sl], 2)
            pltpu.async_copy(
                t4_hbm.at[idx4_v.at[b]], gbufs[b], gsems[b])

        def wait_gather(b):
            pltpu.make_async_copy(
                t4_hbm.at[pl.ds(0, cols_w)], gbufs[b], gsems[b]).wait()

        def fire_scatter(c, b):
            pltpu.async_copy(
                obufs[b].at[:, pl.ds(0, cols_w)],
                out_hbm.at[c, :, pl.ds(col0, cols_w)],
                ssems[b])

        def wait_scatter(b):
            pltpu.make_async_copy(
                obufs[b].at[:, pl.ds(0, cols_w)],
                out_hbm.at[0, :, pl.ds(0, cols_w)],
                ssems[b]).wait()

        lane = lax.iota(jnp.int32, 16)
        dim_rows = [lane + 16 * h for h in range(DIM // 16)]

        def scale(c, b):
            gbuf, obuf = gbufs[b], obufs[b]

            @plsc.parallel_loop(0, cols_w // 16, step=1, unroll=1)
            def _scale(g16):
                l0 = g16 * 16
                qv = lax.bitwise_and(idx_v[c, pl.ds(l0, 16)], 3) * DIM
                for j in range(16):
                    l = l0 + j
                    q = qv[j]
                    lcol = jnp.broadcast_to(l, (16,))
                    for h in range(DIM // 16):
                        vec = gbuf[l, pl.ds(q + 16 * h, 16)]
                        plsc.store_scatter(
                            obuf, [dim_rows[h], lcol], vec * SCALE)

        pltpu.sync_copy(xt_hbm.at[:, pl.ds(col0, cols_w)], idx_v)

        for c in range(GATHER_AHEAD):
            fire_gather(c, c % NBUF)

        @pl.loop(0, n_groups)
        def _group(g):
            for i in range(NBUF):
                c = g * NBUF + i
                wait_gather(i)

                @pl.when(c >= NBUF)
                def _():
                    wait_scatter(i)

                scale(c, i)
                fire_scatter(c, i)

                @pl.when(c + GATHER_AHEAD < B2)
                def _():
                    fire_gather(c + GATHER_AHEAD, (i + GATHER_AHEAD) % NBUF)

        for c in range(B2 - NBUF, B2):
            wait_scatter(c % NBUF)

    return emb_kernel


def kernel(x, table):
    B1, B2 = x.shape
    xt = jnp.transpose(x, (1, 0)).astype(jnp.int32)
    t4 = table.reshape(-1, 128)
    out = _make(B1, B2)(t4, xt)
    return jnp.transpose(out, (2, 0, 1))


# speedup vs baseline: 3.6057x; 1.0638x over previous
import functools
import numpy as np
import jax
import jax.numpy as jnp
from jax import lax
from jax.experimental import pallas as pl
from jax.experimental.pallas import tpu as pltpu
from jax.experimental.pallas import tpu_sc as plsc

DIM = 32
SCALE = np.sqrt(np.float32(DIM)).astype(np.float32)
NC, NS = 2, 16
NW = NC * NS
ROWPACK = 128 // DIM
NBUF = 4
GATHER_AHEAD = 3


@functools.lru_cache(maxsize=None)
def _make(B1, B2):
    cols_w = B1 // NW
    n_groups = B2 // NBUF
    mesh = plsc.VectorSubcoreMesh(
        core_axis_name="c", subcore_axis_name="s",
        num_cores=NC, num_subcores=NS)

    @functools.partial(
        pl.kernel,
        out_type=jax.ShapeDtypeStruct((B2, DIM, B1), jnp.float32),
        mesh=mesh,
        scratch_types=(
            [pltpu.VMEM((B2, cols_w), jnp.int32)]
            + [pltpu.VMEM((cols_w, 128), jnp.float32)] * NBUF
            + [pltpu.VMEM((DIM, cols_w + 1), jnp.float32)] * NBUF
            + [pltpu.SemaphoreType.DMA] * (2 * NBUF)
        ),
        compiler_params=pltpu.CompilerParams(
            use_tc_tiling_on_sc=True, needs_layout_passes=False),
    )
    def emb_kernel(t4_hbm, xt_hbm, out_hbm, idx_v, *scratch):
        gbufs = scratch[:NBUF]
        obufs = scratch[NBUF:2 * NBUF]
        gsems = scratch[2 * NBUF:3 * NBUF]
        ssems = scratch[3 * NBUF:]
        wid = lax.axis_index("s") * NC + lax.axis_index("c")
        col0 = wid * cols_w

        def fire_gather(c, b):
            pltpu.async_copy(
                t4_hbm.at[idx_v.at[c]], gbufs[b], gsems[b])

        def wait_gather(b):
            pltpu.make_async_copy(
                t4_hbm.at[pl.ds(0, cols_w)], gbufs[b], gsems[b]).wait()

        def fire_scatter(c, b):
            pltpu.async_copy(
                obufs[b].at[:, pl.ds(0, cols_w)],
                out_hbm.at[c, :, pl.ds(col0, cols_w)],
                ssems[b])

        def wait_scatter(b):
            pltpu.make_async_copy(
                obufs[b].at[:, pl.ds(0, cols_w)],
                out_hbm.at[0, :, pl.ds(0, cols_w)],
                ssems[b]).wait()

        lane = lax.iota(jnp.int32, 16)
        dim_rows = [lane + 16 * h for h in range(DIM // 16)]

        def scale(c, b):
            gbuf, obuf = gbufs[b], obufs[b]

            @plsc.parallel_loop(0, cols_w, step=1, unroll=4)
            def _scale(l):
                lcol = jnp.broadcast_to(l, (16,))
                for h in range(DIM // 16):
                    vec = gbuf[l, pl.ds(16 * h, 16)]
                    plsc.store_scatter(
                        obuf, [dim_rows[h], lcol], vec * SCALE)

        pltpu.sync_copy(xt_hbm.at[:, pl.ds(col0, cols_w)], idx_v)

        for c in range(GATHER_AHEAD):
            fire_gather(c, c % NBUF)

        @pl.loop(0, n_groups)
        def _group(g):
            for i in range(NBUF):
                c = g * NBUF + i
                wait_gather(i)

                @pl.when(c >= NBUF)
                def _():
                    wait_scatter(i)

                scale(c, i)
                fire_scatter(c, i)

                @pl.when(c + GATHER_AHEAD < B2)
                def _():
                    fire_gather(c + GATHER_AHEAD, (i + GATHER_AHEAD) % NBUF)

        for c in range(B2 - NBUF, B2):
            wait_scatter(c % NBUF)

    return emb_kernel


def kernel(x, table):
    B1, B2 = x.shape
    xt = jnp.transpose(x, (1, 0)).astype(jnp.int32)
    t4 = jnp.pad(table, ((0, 0), (0, 128 - DIM)))
    out = _make(B1, B2)(t4, xt)
    return jnp.transpose(out, (2, 0, 1))


# speedup vs baseline: 5.2820x; 1.4649x over previous
import functools
import numpy as np
import jax
import jax.numpy as jnp
from jax import lax
from jax.experimental import pallas as pl
from jax.experimental.pallas import tpu as pltpu
from jax.experimental.pallas import tpu_sc as plsc

DIM = 32
SCALE = np.sqrt(np.float32(DIM)).astype(np.float32)
NC, NS = 2, 16
NW = NC * NS
NB2 = 2
NBUF = 4
GATHER_AHEAD = 3


@functools.lru_cache(maxsize=None)
def _make(B1, B2):
    cols_w = B1 // NW
    n_chunks = B2 // NB2
    n_groups = n_chunks // NBUF
    assert B2 % NB2 == 0 and n_chunks % NBUF == 0
    mesh = plsc.VectorSubcoreMesh(
        core_axis_name="c", subcore_axis_name="s",
        num_cores=NC, num_subcores=NS)

    @functools.partial(
        pl.kernel,
        out_type=jax.ShapeDtypeStruct((B2, DIM // 8, B1 // 128, 8, 128),
                                      jnp.float32),
        mesh=mesh,
        scratch_types=(
            [pltpu.VMEM((B2, cols_w), jnp.int32)]
            + [pltpu.VMEM((NB2 * cols_w, DIM), jnp.float32)] * NBUF
            + [pltpu.VMEM((NB2, DIM // 8, 8, cols_w + 1), jnp.float32)] * NBUF
            + [pltpu.SemaphoreType.DMA] * (2 * NBUF)
        ),
        compiler_params=pltpu.CompilerParams(
            use_tc_tiling_on_sc=False, needs_layout_passes=False),
    )
    def emb_kernel(table_hbm, xt_hbm, out_hbm, idx_v, *scratch):
        gbufs = scratch[:NBUF]
        obufs = scratch[NBUF:2 * NBUF]
        gsems = scratch[2 * NBUF:3 * NBUF]
        ssems = scratch[3 * NBUF:]
        wid = lax.axis_index("s") * NC + lax.axis_index("c")
        col0 = wid * cols_w

        def fire_gather(c, b):
            for s in range(NB2):
                pltpu.async_copy(
                    table_hbm.at[idx_v.at[c * NB2 + s]],
                    gbufs[b].at[pl.ds(s * cols_w, cols_w)],
                    gsems[b])

        def wait_gather(b):
            pltpu.make_async_copy(
                table_hbm.at[pl.ds(0, NB2 * cols_w)],
                gbufs[b], gsems[b]).wait()

        def fire_scatter(c, b):
            pltpu.async_copy(
                obufs[b].at[:, :, :, pl.ds(0, cols_w)],
                out_hbm.at[pl.ds(c * NB2, NB2), :, wid],
                ssems[b])

        def wait_scatter(b):
            pltpu.make_async_copy(
                obufs[b].at[:, :, :, pl.ds(0, cols_w)],
                out_hbm.at[pl.ds(0, NB2), :, 0],
                ssems[b]).wait()

        lane = lax.iota(jnp.int32, 16)
        dim_rows = [lane + 16 * h for h in range(DIM // 16)]
        big_rows = [lax.div(d, 8) for d in dim_rows]
        sub_rows = [lax.rem(d, 8) for d in dim_rows]
        s_ids = [jnp.broadcast_to(jnp.int32(s), (16,)) for s in range(NB2)]

        def scale(b):
            gbuf, obuf = gbufs[b], obufs[b]

            @plsc.parallel_loop(0, cols_w, step=1, unroll=4)
            def _scale(l):
                lcol = jnp.broadcast_to(l, (16,))
                for s in range(NB2):
                    for h in range(DIM // 16):
                        vec = gbuf[s * cols_w + l, pl.ds(16 * h, 16)]
                        plsc.store_scatter(
                            obuf, [s_ids[s], big_rows[h], sub_rows[h], lcol],
                            vec * SCALE)

        pltpu.sync_copy(xt_hbm.at[:, pl.ds(col0, cols_w)], idx_v)

        for c in range(GATHER_AHEAD):
            fire_gather(c, c % NBUF)

        @pl.loop(0, n_groups)
        def _group(g):
            for i in range(NBUF):
                c = g * NBUF + i
                wait_gather(i)

                @pl.when(c >= NBUF)
                def _():
                    wait_scatter(i)

                scale(i)
                fire_scatter(c, i)

                @pl.when(c + GATHER_AHEAD < n_chunks)
                def _():
                    fire_gather(c + GATHER_AHEAD, (i + GATHER_AHEAD) % NBUF)

        for c in range(n_chunks - NBUF, n_chunks):
            wait_scatter(c % NBUF)

    return emb_kernel


def kernel(x, table):
    B1, B2 = x.shape
    xt = jnp.transpose(x, (1, 0)).astype(jnp.int32)
    out5 = _make(B1, B2)(table, xt)
    out = jnp.transpose(out5, (0, 1, 3, 2, 4)).reshape(B2, DIM, B1)
    return jnp.transpose(out, (2, 0, 1))
